# baseline (speedup 1.0000x reference)
import functools

import jax
import jax.numpy as jnp
from jax import lax
from jax.experimental import pallas as pl
from jax.experimental.pallas import tpu as pltpu
from jax.experimental.pallas import tpu_sc as plsc

B = 4096
C = 256
CP = 257
T = 512
DEPTH = 8
N_INT = 2**DEPTH - 1
N_LEAF = 2**DEPTH
NC = 2
NS = 16
NW = NC * NS
BPW = B // NW
LANES = 16
NV = BPW // LANES
TCHUNK = 8
NCHUNK = T // TCHUNK
N_TOP = 15
TOPW = N_TOP * LANES


def _forest_body(x_hbm, th_hbm, or_hbm, w_hbm, tt_hbm, to_hbm, out_hbm,
                 x_v, th_a, or_a, w_a, tt_a, to_a,
                 th_b, or_b, w_b, tt_b, to_b, out_v,
                 sem_a, sem_b):
    wid = lax.axis_index("s") * NC + lax.axis_index("c")
    b0 = wid * BPW
    pltpu.sync_copy(x_hbm.at[pl.ds(b0 * CP, BPW * CP)], x_v)

    lane_iota = lax.iota(jnp.int32, LANES)
    zeros = jnp.zeros((LANES,), jnp.int32)
    xbases = [(lane_iota + (i * LANES)) * CP for i in range(NV)]

    def fetch(c, th_v, or_v, w_v, tt_v, to_v, sem):
        off = c * (TCHUNK * N_LEAF)
        offt = c * (TCHUNK * TOPW)
        pltpu.async_copy(th_hbm.at[pl.ds(off, TCHUNK * N_LEAF)], th_v, sem)
        pltpu.async_copy(or_hbm.at[pl.ds(off, TCHUNK * N_LEAF)], or_v, sem)
        pltpu.async_copy(w_hbm.at[pl.ds(off, TCHUNK * N_LEAF)], w_v, sem)
        pltpu.async_copy(tt_hbm.at[pl.ds(offt, TCHUNK * TOPW)], tt_v, sem)
        pltpu.async_copy(to_hbm.at[pl.ds(offt, TCHUNK * TOPW)], to_v, sem)

    def drain(th_v, or_v, w_v, tt_v, to_v, sem):
        pltpu.make_async_copy(
            th_hbm.at[pl.ds(0, TCHUNK * N_LEAF)], th_v, sem).wait()
        pltpu.make_async_copy(
            or_hbm.at[pl.ds(0, TCHUNK * N_LEAF)], or_v, sem).wait()
        pltpu.make_async_copy(
            w_hbm.at[pl.ds(0, TCHUNK * N_LEAF)], w_v, sem).wait()
        pltpu.make_async_copy(
            tt_hbm.at[pl.ds(0, TCHUNK * TOPW)], tt_v, sem).wait()
        pltpu.make_async_copy(
            to_hbm.at[pl.ds(0, TCHUNK * TOPW)], to_v, sem).wait()

    def compute_chunk(c, th_v, or_v, w_v, tt_v, to_v):
        t0 = c * TCHUNK

        def tree_body(tc, carry2):
            tb = tc * N_LEAF
            tbase = tc * TOPW
            obase = lane_iota + (t0 + tc) * BPW
            ta0 = lane_iota + tbase
            k1 = ta0 + LANES
            k2 = k1 + LANES
            ct0 = LANES - ta0
            ct1 = ct0 + LANES
            a1 = zeros + (tb + 1)
            a2 = a1 + 1
            cv0 = zeros + (1 - tb)
            cv1 = cv0 + 1
            cw0 = cv0 - N_INT
            cw1 = cw0 + 1
            th0 = plsc.load_gather(tt_v, [ta0])
            od0 = plsc.load_gather(to_v, [ta0])
            fts = [plsc.load_gather(x_v, [xbases[i] + od0])
                   for i in range(NV)]
            ms = [fts[i] - th0 for i in range(NV)]
            mabs = [jnp.abs(ms[i]) for i in range(NV)]
            nas = [jnp.where(ms[i] > 0, k2, k1) for i in range(NV)]
            for _ in range(2):
                ths = [plsc.load_gather(tt_v, [nas[i]]) for i in range(NV)]
                ods = [plsc.load_gather(to_v, [nas[i]]) for i in range(NV)]
                fts = [plsc.load_gather(x_v, [xbases[i] + ods[i]])
                       for i in range(NV)]
                for i in range(NV):
                    m = fts[i] - ths[i]
                    mabs[i] = jnp.minimum(mabs[i], jnp.abs(m))
                    nas[i] = (nas[i] + nas[i]) + jnp.where(m > 0, ct1, ct0)
            ths = [plsc.load_gather(tt_v, [nas[i]]) for i in range(NV)]
            ods = [plsc.load_gather(to_v, [nas[i]]) for i in range(NV)]
            fts = [plsc.load_gather(x_v, [xbases[i] + ods[i]])
                   for i in range(NV)]
            for i in range(NV):
                m = fts[i] - ths[i]
                mabs[i] = jnp.minimum(mabs[i], jnp.abs(m))
                nas[i] = ((nas[i] - ta0) >> 3) + jnp.where(m > 0, a2, a1)
            for _ in range(3):
                ths = [plsc.load_gather(th_v, [nas[i]]) for i in range(NV)]
                ods = [plsc.load_gather(or_v, [nas[i]]) for i in range(NV)]
                fts = [plsc.load_gather(x_v, [xbases[i] + ods[i]])
                       for i in range(NV)]
                for i in range(NV):
                    m = fts[i] - ths[i]
                    mabs[i] = jnp.minimum(mabs[i], jnp.abs(m))
                    nas[i] = (nas[i] + nas[i]) + jnp.where(m > 0, cv1, cv0)
            ths = [plsc.load_gather(th_v, [nas[i]]) for i in range(NV)]
            ods = [plsc.load_gather(or_v, [nas[i]]) for i in range(NV)]
            fts = [plsc.load_gather(x_v, [xbases[i] + ods[i]])
                   for i in range(NV)]
            for i in range(NV):
                m = fts[i] - ths[i]
                mabs[i] = jnp.minimum(mabs[i], jnp.abs(m))
                nas[i] = (nas[i] + nas[i]) + jnp.where(m > 0, cw1, cw0)
            ws = [plsc.load_gather(w_v, [nas[i]]) for i in range(NV)]
            for i in range(NV):
                plsc.store_scatter(out_v, [obase + (i * LANES)],
                                   ws[i] * mabs[i])
            return carry2

        lax.fori_loop(0, TCHUNK, tree_body, 0)

    fetch(0, th_a, or_a, w_a, tt_a, to_a, sem_a)
    fetch(1, th_b, or_b, w_b, tt_b, to_b, sem_b)

    def pair_body(i, carry):
        c = 2 * i
        drain(th_a, or_a, w_a, tt_a, to_a, sem_a)
        compute_chunk(c, th_a, or_a, w_a, tt_a, to_a)
        fetch(jnp.minimum(c + 2, NCHUNK - 1), th_a, or_a, w_a, tt_a, to_a,
              sem_a)
        drain(th_b, or_b, w_b, tt_b, to_b, sem_b)
        compute_chunk(c + 1, th_b, or_b, w_b, tt_b, to_b)
        fetch(jnp.minimum(c + 3, NCHUNK - 1), th_b, or_b, w_b, tt_b, to_b,
              sem_b)
        return carry

    lax.fori_loop(0, NCHUNK // 2, pair_body, 0)
    drain(th_a, or_a, w_a, tt_a, to_a, sem_a)
    drain(th_b, or_b, w_b, tt_b, to_b, sem_b)
    pltpu.sync_copy(out_v, out_hbm.at[pl.ds(b0 * T, BPW * T)])


@jax.jit
def _forest(x, th_pad, or_pad, weights, top_th, top_or):
    mesh = plsc.VectorSubcoreMesh(core_axis_name="c", subcore_axis_name="s")
    fwd = functools.partial(
        pl.kernel,
        mesh=mesh,
        compiler_params=pltpu.CompilerParams(
            use_tc_tiling_on_sc=False, needs_layout_passes=False),
        out_type=jax.ShapeDtypeStruct((B * T,), jnp.float32),
        scratch_types=[
            pltpu.VMEM((BPW * CP,), jnp.float32),
            pltpu.VMEM((TCHUNK * N_LEAF,), jnp.float32),
            pltpu.VMEM((TCHUNK * N_LEAF,), jnp.int32),
            pltpu.VMEM((TCHUNK * N_LEAF,), jnp.float32),
            pltpu.VMEM((TCHUNK * TOPW,), jnp.float32),
            pltpu.VMEM((TCHUNK * TOPW,), jnp.int32),
            pltpu.VMEM((TCHUNK * N_LEAF,), jnp.float32),
            pltpu.VMEM((TCHUNK * N_LEAF,), jnp.int32),
            pltpu.VMEM((TCHUNK * N_LEAF,), jnp.float32),
            pltpu.VMEM((TCHUNK * TOPW,), jnp.float32),
            pltpu.VMEM((TCHUNK * TOPW,), jnp.int32),
            pltpu.VMEM((BPW * T,), jnp.float32),
            pltpu.SemaphoreType.DMA,
            pltpu.SemaphoreType.DMA,
        ],
    )(_forest_body)
    out = fwd(x.reshape(-1), th_pad.reshape(-1), or_pad.reshape(-1),
              weights.reshape(-1), top_th.reshape(-1), top_or.reshape(-1))
    return out.reshape(NW, T, BPW).transpose(0, 2, 1).reshape(B, T)


def kernel(x, thresholds, weights, ordinals):
    th_pad = jnp.pad(thresholds, ((0, 0), (0, 1)))
    or_pad = jnp.pad(ordinals, ((0, 0), (0, 1)))
    x_pad = jnp.pad(x, ((0, 0), (0, CP - C)))
    top_th = jnp.broadcast_to(
        thresholds[:, :N_TOP, None], (T, N_TOP, LANES)).reshape(T, TOPW)
    top_or = jnp.broadcast_to(
        ordinals[:, :N_TOP, None], (T, N_TOP, LANES)).reshape(T, TOPW)
    return _forest(x_pad, th_pad, or_pad, weights, top_th, top_or)


# speedup vs baseline: 1.1374x; 1.1374x over previous
import functools

import jax
import jax.numpy as jnp
from jax import lax
from jax.experimental import pallas as pl
from jax.experimental.pallas import tpu as pltpu
from jax.experimental.pallas import tpu_sc as plsc

B = 4096
C = 256
CP = 257
T = 512
DEPTH = 8
N_INT = 2**DEPTH - 1
N_LEAF = 2**DEPTH
NC = 2
NS = 16
NW = NC * NS
BPW = B // NW
LANES = 16
NV = BPW // LANES
TCHUNK = 8
NCHUNK = T // TCHUNK


def _forest_body(x_hbm, th_hbm, or_hbm, w_hbm, out_hbm,
                 x_v, th_a, or_a, w_a, th_b, or_b, w_b, out_v,
                 sem_a, sem_b):
    wid = lax.axis_index("s") * NC + lax.axis_index("c")
    b0 = wid * BPW
    pltpu.sync_copy(x_hbm.at[pl.ds(b0 * CP, BPW * CP)], x_v)

    lane_iota = lax.iota(jnp.int32, LANES)
    zeros = jnp.zeros((LANES,), jnp.int32)
    xbases = [(lane_iota + (i * LANES)) * CP for i in range(NV)]

    def fetch(c, th_v, or_v, w_v, sem):
        off = c * (TCHUNK * N_LEAF)
        pltpu.async_copy(th_hbm.at[pl.ds(off, TCHUNK * N_LEAF)], th_v, sem)
        pltpu.async_copy(or_hbm.at[pl.ds(off, TCHUNK * N_LEAF)], or_v, sem)
        pltpu.async_copy(w_hbm.at[pl.ds(off, TCHUNK * N_LEAF)], w_v, sem)

    def drain(th_v, or_v, w_v, sem):
        pltpu.make_async_copy(
            th_hbm.at[pl.ds(0, TCHUNK * N_LEAF)], th_v, sem).wait()
        pltpu.make_async_copy(
            or_hbm.at[pl.ds(0, TCHUNK * N_LEAF)], or_v, sem).wait()
        pltpu.make_async_copy(
            w_hbm.at[pl.ds(0, TCHUNK * N_LEAF)], w_v, sem).wait()

    def compute_chunk(c, th_v, or_v, w_v):
        t0 = c * TCHUNK

        def tree_body(tc, carry2):
            tb = tc * N_LEAF
            obase = lane_iota + (t0 + tc) * BPW
            cv0 = zeros + (1 - tb)
            cv1 = cv0 + 1
            cw0 = cv0 - N_INT
            cw1 = cw0 + 1
            k1 = zeros + (tb + 1)
            k2 = k1 + 1
            th0 = plsc.load_gather(th_v, [zeros + tb])
            od0 = plsc.load_gather(or_v, [zeros + tb])
            fts = [plsc.load_gather(x_v, [xbases[i] + od0])
                   for i in range(NV)]
            ms = [fts[i] - th0 for i in range(NV)]
            mabs = [jnp.abs(ms[i]) for i in range(NV)]
            nas = [jnp.where(ms[i] > 0, k2, k1) for i in range(NV)]
            for _ in range(DEPTH - 2):
                ths = [plsc.load_gather(th_v, [nas[i]]) for i in range(NV)]
                ods = [plsc.load_gather(or_v, [nas[i]]) for i in range(NV)]
                fts = [plsc.load_gather(x_v, [xbases[i] + ods[i]])
                       for i in range(NV)]
                for i in range(NV):
                    m = fts[i] - ths[i]
                    mabs[i] = jnp.minimum(mabs[i], jnp.abs(m))
                    nas[i] = (nas[i] + nas[i]) + jnp.where(m > 0, cv1, cv0)
            ths = [plsc.load_gather(th_v, [nas[i]]) for i in range(NV)]
            ods = [plsc.load_gather(or_v, [nas[i]]) for i in range(NV)]
            fts = [plsc.load_gather(x_v, [xbases[i] + ods[i]])
                   for i in range(NV)]
            for i in range(NV):
                m = fts[i] - ths[i]
                mabs[i] = jnp.minimum(mabs[i], jnp.abs(m))
                nas[i] = (nas[i] + nas[i]) + jnp.where(m > 0, cw1, cw0)
            ws = [plsc.load_gather(w_v, [nas[i]]) for i in range(NV)]
            for i in range(NV):
                plsc.store_scatter(out_v, [obase + (i * LANES)],
                                   ws[i] * mabs[i])
            return carry2

        lax.fori_loop(0, TCHUNK, tree_body, 0)

    fetch(0, th_a, or_a, w_a, sem_a)
    fetch(1, th_b, or_b, w_b, sem_b)

    def pair_body(i, carry):
        c = 2 * i
        drain(th_a, or_a, w_a, sem_a)
        compute_chunk(c, th_a, or_a, w_a)
        fetch(jnp.minimum(c + 2, NCHUNK - 1), th_a, or_a, w_a, sem_a)
        drain(th_b, or_b, w_b, sem_b)
        compute_chunk(c + 1, th_b, or_b, w_b)
        fetch(jnp.minimum(c + 3, NCHUNK - 1), th_b, or_b, w_b, sem_b)
        return carry

    lax.fori_loop(0, NCHUNK // 2, pair_body, 0)
    drain(th_a, or_a, w_a, sem_a)
    drain(th_b, or_b, w_b, sem_b)
    pltpu.sync_copy(out_v, out_hbm.at[pl.ds(b0 * T, BPW * T)])


@jax.jit
def _forest(x, th_pad, or_pad, weights):
    mesh = plsc.VectorSubcoreMesh(core_axis_name="c", subcore_axis_name="s")
    fwd = functools.partial(
        pl.kernel,
        mesh=mesh,
        compiler_params=pltpu.CompilerParams(
            use_tc_tiling_on_sc=False, needs_layout_passes=False),
        out_type=jax.ShapeDtypeStruct((B * T,), jnp.float32),
        scratch_types=[
            pltpu.VMEM((BPW * CP,), jnp.float32),
            pltpu.VMEM((TCHUNK * N_LEAF,), jnp.float32),
            pltpu.VMEM((TCHUNK * N_LEAF,), jnp.int32),
            pltpu.VMEM((TCHUNK * N_LEAF,), jnp.float32),
            pltpu.VMEM((TCHUNK * N_LEAF,), jnp.float32),
            pltpu.VMEM((TCHUNK * N_LEAF,), jnp.int32),
            pltpu.VMEM((TCHUNK * N_LEAF,), jnp.float32),
            pltpu.VMEM((BPW * T,), jnp.float32),
            pltpu.SemaphoreType.DMA,
            pltpu.SemaphoreType.DMA,
        ],
    )(_forest_body)
    out = fwd(x.reshape(-1), th_pad.reshape(-1), or_pad.reshape(-1),
              weights.reshape(-1))
    return out.reshape(NW, T, BPW).transpose(0, 2, 1).reshape(B, T)


def kernel(x, thresholds, weights, ordinals):
    th_pad = jnp.pad(thresholds, ((0, 0), (0, 1)))
    or_pad = jnp.pad(ordinals, ((0, 0), (0, 1)))
    x_pad = jnp.pad(x, ((0, 0), (0, CP - C)))
    return _forest(x_pad, th_pad, or_pad, weights)


# speedup vs baseline: 1.1534x; 1.0141x over previous
import functools

import jax
import jax.numpy as jnp
from jax import lax
from jax.experimental import pallas as pl
from jax.experimental.pallas import tpu as pltpu
from jax.experimental.pallas import tpu_sc as plsc

B = 4096
C = 256
CP = 257
T = 512
DEPTH = 8
N_INT = 2**DEPTH - 1
N_LEAF = 2**DEPTH
NC = 2
NS = 16
NW = NC * NS
BPW = B // NW
LANES = 16
NV = BPW // LANES
TCHUNK = 16
NCHUNK = T // TCHUNK


def _forest_body(x_hbm, th_hbm, or_hbm, w_hbm, out_hbm,
                 x_v, th_a, or_a, w_a, th_b, or_b, w_b, out_v,
                 sem_a, sem_b):
    wid = lax.axis_index("s") * NC + lax.axis_index("c")
    b0 = wid * BPW
    pltpu.sync_copy(x_hbm.at[pl.ds(b0 * CP, BPW * CP)], x_v)

    lane_iota = lax.iota(jnp.int32, LANES)
    zeros = jnp.zeros((LANES,), jnp.int32)
    xbases = [(lane_iota + (i * LANES)) * CP for i in range(NV)]

    def fetch(c, th_v, or_v, w_v, sem):
        off = c * (TCHUNK * N_LEAF)
        pltpu.async_copy(th_hbm.at[pl.ds(off, TCHUNK * N_LEAF)], th_v, sem)
        pltpu.async_copy(or_hbm.at[pl.ds(off, TCHUNK * N_LEAF)], or_v, sem)
        pltpu.async_copy(w_hbm.at[pl.ds(off, TCHUNK * N_LEAF)], w_v, sem)

    def drain(th_v, or_v, w_v, sem):
        pltpu.make_async_copy(
            th_hbm.at[pl.ds(0, TCHUNK * N_LEAF)], th_v, sem).wait()
        pltpu.make_async_copy(
            or_hbm.at[pl.ds(0, TCHUNK * N_LEAF)], or_v, sem).wait()
        pltpu.make_async_copy(
            w_hbm.at[pl.ds(0, TCHUNK * N_LEAF)], w_v, sem).wait()

    def compute_chunk(c, th_v, or_v, w_v):
        t0 = c * TCHUNK

        def one_tree(tc):
            tb = tc * N_LEAF
            obase = lane_iota + (t0 + tc) * BPW
            cv0 = zeros + (1 - tb)
            cv1 = cv0 + 1
            cw0 = cv0 - N_INT
            cw1 = cw0 + 1
            k1 = zeros + (tb + 1)
            k2 = k1 + 1
            th0 = plsc.load_gather(th_v, [zeros + tb])
            od0 = plsc.load_gather(or_v, [zeros + tb])
            fts = [plsc.load_gather(x_v, [xbases[i] + od0])
                   for i in range(NV)]
            ms = [fts[i] - th0 for i in range(NV)]
            mabs = [jnp.abs(ms[i]) for i in range(NV)]
            nas = [jnp.where(ms[i] > 0, k2, k1) for i in range(NV)]
            for _ in range(DEPTH - 2):
                ths = [plsc.load_gather(th_v, [nas[i]]) for i in range(NV)]
                ods = [plsc.load_gather(or_v, [nas[i]]) for i in range(NV)]
                fts = [plsc.load_gather(x_v, [xbases[i] + ods[i]])
                       for i in range(NV)]
                for i in range(NV):
                    m = fts[i] - ths[i]
                    mabs[i] = jnp.minimum(mabs[i], jnp.abs(m))
                    nas[i] = (nas[i] + nas[i]) + jnp.where(m > 0, cv1, cv0)
            ths = [plsc.load_gather(th_v, [nas[i]]) for i in range(NV)]
            ods = [plsc.load_gather(or_v, [nas[i]]) for i in range(NV)]
            fts = [plsc.load_gather(x_v, [xbases[i] + ods[i]])
                   for i in range(NV)]
            for i in range(NV):
                m = fts[i] - ths[i]
                mabs[i] = jnp.minimum(mabs[i], jnp.abs(m))
                nas[i] = (nas[i] + nas[i]) + jnp.where(m > 0, cw1, cw0)
            ws = [plsc.load_gather(w_v, [nas[i]]) for i in range(NV)]
            for i in range(NV):
                plsc.store_scatter(out_v, [obase + (i * LANES)],
                                   ws[i] * mabs[i])

        def tree_body(p, carry2):
            one_tree(2 * p)
            one_tree(2 * p + 1)
            return carry2

        lax.fori_loop(0, TCHUNK // 2, tree_body, 0)

    fetch(0, th_a, or_a, w_a, sem_a)
    fetch(1, th_b, or_b, w_b, sem_b)

    def pair_body(i, carry):
        c = 2 * i
        drain(th_a, or_a, w_a, sem_a)
        compute_chunk(c, th_a, or_a, w_a)
        fetch(jnp.minimum(c + 2, NCHUNK - 1), th_a, or_a, w_a, sem_a)
        drain(th_b, or_b, w_b, sem_b)
        compute_chunk(c + 1, th_b, or_b, w_b)
        fetch(jnp.minimum(c + 3, NCHUNK - 1), th_b, or_b, w_b, sem_b)
        return carry

    lax.fori_loop(0, NCHUNK // 2, pair_body, 0)
    drain(th_a, or_a, w_a, sem_a)
    drain(th_b, or_b, w_b, sem_b)
    pltpu.sync_copy(out_v, out_hbm.at[pl.ds(b0 * T, BPW * T)])


@jax.jit
def _forest(x, th_pad, or_pad, weights):
    mesh = plsc.VectorSubcoreMesh(core_axis_name="c", subcore_axis_name="s")
    fwd = functools.partial(
        pl.kernel,
        mesh=mesh,
        compiler_params=pltpu.CompilerParams(
            use_tc_tiling_on_sc=False, needs_layout_passes=False),
        out_type=jax.ShapeDtypeStruct((B * T,), jnp.float32),
        scratch_types=[
            pltpu.VMEM((BPW * CP,), jnp.float32),
            pltpu.VMEM((TCHUNK * N_LEAF,), jnp.float32),
            pltpu.VMEM((TCHUNK * N_LEAF,), jnp.int32),
            pltpu.VMEM((TCHUNK * N_LEAF,), jnp.float32),
            pltpu.VMEM((TCHUNK * N_LEAF,), jnp.float32),
            pltpu.VMEM((TCHUNK * N_LEAF,), jnp.int32),
            pltpu.VMEM((TCHUNK * N_LEAF,), jnp.float32),
            pltpu.VMEM((BPW * T,), jnp.float32),
            pltpu.SemaphoreType.DMA,
            pltpu.SemaphoreType.DMA,
        ],
    )(_forest_body)
    out = fwd(x.reshape(-1), th_pad.reshape(-1), or_pad.reshape(-1),
              weights.reshape(-1))
    return out.reshape(NW, T, BPW).transpose(0, 2, 1).reshape(B, T)


def kernel(x, thresholds, weights, ordinals):
    th_pad = jnp.pad(thresholds, ((0, 0), (0, 1)))
    or_pad = jnp.pad(ordinals, ((0, 0), (0, 1)))
    x_pad = jnp.pad(x, ((0, 0), (0, CP - C)))
    return _forest(x_pad, th_pad, or_pad, weights)
